# baseline (speedup 1.0000x reference)
import jax
import jax.numpy as jnp
from jax import lax
from jax.experimental import pallas as pl
from jax.experimental.pallas import tpu as pltpu
from jax.experimental.pallas import tpu_sc as plsc

B, L, NBINS, D = 4, 512, 65, 128
NC, NS, LANES = 2, 16, 16
NW = NC * NS
ROWS = B * L * L
NP = B * L
NP_TC = 1024
RTC = NP_TC * L
SC_ROWS = ROWS - RTC
SC_ROWS_PER_W = SC_ROWS // NW
CHUNK = 256
TCBLK = 1024


def _sc_body(table_hbm, res_hbm, out_hbm,
             table_v, res_v, res_s, rows0, rows1, osem0, osem1):
    wid = lax.axis_index("s") * NC + lax.axis_index("c")
    pltpu.sync_copy(table_hbm, table_v)
    bw = lax.shift_right_logical(RTC + wid * SC_ROWS_PER_W, 18)
    pltpu.sync_copy(res_hbm.at[pl.ds(bw * L, L)], res_v)

    def stage_smem(g, _):
        blk = res_v[pl.ds(g * LANES, LANES)]
        for t in range(LANES):
            res_s[g * LANES + t] = blk[t]
        return ()

    lax.fori_loop(0, L // LANES, stage_smem, ())
    rows = (rows0, rows1)
    osem = (osem0, osem1)

    def chunk_pair_body(c2, _):
        for p in range(2):
            c = c2 * 2 + p
            r0 = RTC + wid * SC_ROWS_PER_W + c * CHUNK
            iloc = jnp.bitwise_and(lax.shift_right_logical(r0, 9), L - 1)
            j0 = jnp.bitwise_and(r0, L - 1)
            ri = res_s[iloc]

            @pl.when(c2 > 0)
            def _wait_prev():
                pltpu.make_async_copy(
                    rows[p], out_hbm.at[pl.ds(0, CHUNK * D)], osem[p]).wait()

            @plsc.parallel_loop(0, CHUNK, unroll=4)
            def row_body(r):
                rj = res_s[j0 + r]
                src = lax.shift_left(
                    lax.min(lax.max(ri - rj, -32), 32) + 32, 7)
                dst = lax.shift_left(r, 7)
                for k in range(D // LANES):
                    rows[p][pl.ds(dst + k * LANES, LANES)] = \
                        table_v[pl.ds(src + k * LANES, LANES)]

            pltpu.async_copy(
                rows[p], out_hbm.at[pl.ds((r0 - RTC) * D, CHUNK * D)], osem[p])
        return ()

    lax.fori_loop(0, SC_ROWS_PER_W // CHUNK // 2, chunk_pair_body, ())
    for p in range(2):
        pltpu.make_async_copy(
            rows[p], out_hbm.at[pl.ds(0, CHUNK * D)], osem[p]).wait()


def _tc_body(ri_ref, rj_ref, tab_ref, o_ref):
    d = ri_ref[0, 0, :] - rj_ref[0, 0, :]
    dc = jnp.clip(d, -32, 32) + 32
    oh = (dc[:, None]
          == lax.broadcasted_iota(jnp.int32, (1, NBINS), 1)).astype(jnp.float32)
    o_ref[...] = jnp.dot(oh, tab_ref[...], preferred_element_type=jnp.float32)


@jax.jit
def kernel(res_id, W, b):
    table = W.T + b[None, :]
    res_flat = res_id.reshape(-1).astype(jnp.int32)
    mesh = plsc.VectorSubcoreMesh(
        core_axis_name="c", subcore_axis_name="s", num_cores=NC, num_subcores=NS
    )
    sc_out = pl.kernel(
        _sc_body,
        out_type=jax.ShapeDtypeStruct((SC_ROWS * D,), jnp.float32),
        mesh=mesh,
        compiler_params=pltpu.CompilerParams(needs_layout_passes=False),
        scratch_types=[
            pltpu.VMEM((NBINS * D,), jnp.float32),
            pltpu.VMEM((L,), jnp.int32),
            pltpu.SMEM((L,), jnp.int32),
            pltpu.VMEM((CHUNK * D,), jnp.float32),
            pltpu.VMEM((CHUNK * D,), jnp.float32),
            pltpu.SemaphoreType.DMA,
            pltpu.SemaphoreType.DMA,
        ],
    )(table.reshape(-1), res_flat)

    nblk = RTC // TCBLK
    ri_row = jnp.repeat(res_flat[:NP_TC], L).reshape(nblk, 1, TCBLK)
    rj_row = jnp.broadcast_to(
        res_id.astype(jnp.int32)[:, None, :], (B, L, L)
    ).reshape(-1)[:RTC].reshape(nblk, 1, TCBLK)
    tc_out = pl.pallas_call(
        _tc_body,
        grid=(nblk,),
        in_specs=[
            pl.BlockSpec((1, 1, TCBLK), lambda i: (i, 0, 0)),
            pl.BlockSpec((1, 1, TCBLK), lambda i: (i, 0, 0)),
            pl.BlockSpec((NBINS, D), lambda i: (0, 0)),
        ],
        out_specs=pl.BlockSpec((TCBLK, D), lambda i: (i, 0)),
        out_shape=jax.ShapeDtypeStruct((RTC, D), jnp.float32),
        compiler_params=pltpu.CompilerParams(
            dimension_semantics=("arbitrary",)),
    )(ri_row, rj_row, table)

    out = jnp.concatenate([tc_out.reshape(-1), sc_out])
    return out.reshape(B, L, L, D)


# speedup vs baseline: 3.9844x; 3.9844x over previous
import jax
import jax.numpy as jnp
from jax import lax
from jax.experimental import pallas as pl
from jax.experimental.pallas import tpu as pltpu
from jax.experimental.pallas import tpu_sc as plsc

B, L, NBINS, D = 4, 512, 65, 128
NC, NS, LANES = 2, 16, 16
NW = NC * NS
ROWS = B * L * L
ROWS_PER_W = ROWS // NW
CHUNK = 256
CHUNKS_PER_W = ROWS_PER_W // CHUNK


def _relpos_body(table_hbm, res_hbm, out_hbm,
                 table_v, res_v, res_s, rows0, rows1, osem0, osem1, tsem):
    wid = lax.axis_index("s") * NC + lax.axis_index("c")
    tcopy = pltpu.async_copy(table_hbm, table_v, tsem)
    pltpu.sync_copy(res_hbm.at[pl.ds((wid // (NW // B)) * L, L)], res_v)

    def stage_smem(g, _):
        blk = res_v[pl.ds(g * LANES, LANES)]
        for t in range(LANES):
            res_s[g * LANES + t] = blk[t]
        return ()

    lax.fori_loop(0, L // LANES, stage_smem, ())
    tcopy.wait()
    rows = (rows0, rows1)
    osem = (osem0, osem1)

    def chunk_pair_body(c2, _):
        for p in range(2):
            c = c2 * 2 + p
            r0 = wid * ROWS_PER_W + c * CHUNK
            iloc = jnp.bitwise_and(lax.shift_right_logical(r0, 9), L - 1)
            j0 = jnp.bitwise_and(r0, L - 1)
            ri = res_s[iloc]

            @pl.when(c2 > 0)
            def _wait_prev():
                pltpu.make_async_copy(
                    rows[p], out_hbm.at[pl.ds(0, CHUNK * D)], osem[p]).wait()

            @plsc.parallel_loop(0, CHUNK, unroll=4)
            def row_body(r):
                rj = res_s[j0 + r]
                src = lax.shift_left(
                    lax.min(lax.max(ri - rj, -32), 32) + 32, 7)
                dst = lax.shift_left(r, 7)
                for k in range(D // LANES):
                    rows[p][pl.ds(dst + k * LANES, LANES)] = \
                        table_v[pl.ds(src + k * LANES, LANES)]

            pltpu.async_copy(
                rows[p], out_hbm.at[pl.ds(r0 * D, CHUNK * D)], osem[p])
        return ()

    lax.fori_loop(0, CHUNKS_PER_W // 2, chunk_pair_body, ())
    for p in range(2):
        pltpu.make_async_copy(
            rows[p], out_hbm.at[pl.ds(0, CHUNK * D)], osem[p]).wait()


@jax.jit
def kernel(res_id, W, b):
    table = (W.T + b[None, :]).reshape(-1)
    res_flat = res_id.reshape(-1).astype(jnp.int32)
    mesh = plsc.VectorSubcoreMesh(
        core_axis_name="c", subcore_axis_name="s", num_cores=NC, num_subcores=NS
    )
    out = pl.kernel(
        _relpos_body,
        out_type=jax.ShapeDtypeStruct((ROWS * D,), jnp.float32),
        mesh=mesh,
        compiler_params=pltpu.CompilerParams(needs_layout_passes=False),
        scratch_types=[
            pltpu.VMEM((NBINS * D,), jnp.float32),
            pltpu.VMEM((L,), jnp.int32),
            pltpu.SMEM((L,), jnp.int32),
            pltpu.VMEM((CHUNK * D,), jnp.float32),
            pltpu.VMEM((CHUNK * D,), jnp.float32),
            pltpu.SemaphoreType.DMA,
            pltpu.SemaphoreType.DMA,
            pltpu.SemaphoreType.DMA,
        ],
    )(table, res_flat)
    return out.reshape(B, L, L, D)


# speedup vs baseline: 4.0443x; 1.0150x over previous
import jax
import jax.numpy as jnp
from jax import lax
from jax.experimental import pallas as pl
from jax.experimental.pallas import tpu as pltpu
from jax.experimental.pallas import tpu_sc as plsc

B, L, NBINS, D = 4, 512, 65, 128
NC, NS, LANES = 2, 16, 16
NW = NC * NS
ROWS = B * L * L
ROWS_PER_W = ROWS // NW
CHUNK = 256
CHUNKS_PER_W = ROWS_PER_W // CHUNK


def _relpos_body(table_hbm, res_hbm, out_hbm,
                 table_v, res_v, res_s, rows0, rows1, osem0, osem1, tsem):
    wid = lax.axis_index("s") * NC + lax.axis_index("c")
    tcopy = pltpu.async_copy(table_hbm, table_v, tsem)
    pltpu.sync_copy(res_hbm.at[pl.ds((wid // (NW // B)) * L, L)], res_v)

    def stage_smem(g, _):
        blk = res_v[pl.ds(g * LANES, LANES)]
        for t in range(LANES):
            res_s[g * LANES + t] = blk[t]
        return ()

    lax.fori_loop(0, L // LANES, stage_smem, ())
    tcopy.wait()
    rows = (rows0, rows1)
    osem = (osem0, osem1)

    def chunk_pair_body(c2, _):
        for p in range(2):
            c = c2 * 2 + p
            r0 = wid * ROWS_PER_W + c * CHUNK
            iloc = jnp.bitwise_and(lax.shift_right_logical(r0, 9), L - 1)
            j0 = jnp.bitwise_and(r0, L - 1)
            ri = res_s[iloc]

            @pl.when(c2 > 0)
            def _wait_prev():
                pltpu.make_async_copy(
                    rows[p], out_hbm.at[pl.ds(0, CHUNK * D)], osem[p]).wait()

            @plsc.parallel_loop(0, CHUNK, unroll=2)
            def row_body(r):
                rj = res_s[j0 + r]
                src = lax.shift_left(
                    lax.min(lax.max(ri - rj, -32), 32) + 32, 7)
                dst = lax.shift_left(r, 7)
                for k in range(D // LANES):
                    rows[p][pl.ds(dst + k * LANES, LANES)] = \
                        table_v[pl.ds(src + k * LANES, LANES)]

            pltpu.async_copy(
                rows[p], out_hbm.at[pl.ds(r0 * D, CHUNK * D)], osem[p])
        return ()

    lax.fori_loop(0, CHUNKS_PER_W // 2, chunk_pair_body, ())
    for p in range(2):
        pltpu.make_async_copy(
            rows[p], out_hbm.at[pl.ds(0, CHUNK * D)], osem[p]).wait()


@jax.jit
def kernel(res_id, W, b):
    table = (W.T + b[None, :]).reshape(-1)
    res_flat = res_id.reshape(-1).astype(jnp.int32)
    mesh = plsc.VectorSubcoreMesh(
        core_axis_name="c", subcore_axis_name="s", num_cores=NC, num_subcores=NS
    )
    out = pl.kernel(
        _relpos_body,
        out_type=jax.ShapeDtypeStruct((ROWS * D,), jnp.float32),
        mesh=mesh,
        compiler_params=pltpu.CompilerParams(needs_layout_passes=False),
        scratch_types=[
            pltpu.VMEM((NBINS * D,), jnp.float32),
            pltpu.VMEM((L,), jnp.int32),
            pltpu.SMEM((L,), jnp.int32),
            pltpu.VMEM((CHUNK * D,), jnp.float32),
            pltpu.VMEM((CHUNK * D,), jnp.float32),
            pltpu.SemaphoreType.DMA,
            pltpu.SemaphoreType.DMA,
            pltpu.SemaphoreType.DMA,
        ],
    )(table, res_flat)
    return out.reshape(B, L, L, D)
